# Initial kernel scaffold; baseline (speedup 1.0000x reference)
import jax
import jax.numpy as jnp
from jax.experimental import pallas as pl


def kernel(x, E):
    raise NotImplementedError("write your pallas kernel here")


# speedup vs baseline: 1.1113x; 1.1113x over previous
import functools

import jax
import jax.numpy as jnp
from jax import lax
from jax.experimental import pallas as pl
from jax.experimental.pallas import tpu as pltpu
from jax.experimental.pallas import tpu_sc as plsc

DIM = 32
NUM_CORES = 2
NUM_SUBCORES = 16
NW = NUM_CORES * NUM_SUBCORES
CHUNK = 3200


@functools.partial(
    pl.kernel,
    mesh=plsc.VectorSubcoreMesh(core_axis_name="c", subcore_axis_name="s"),
    out_type=jax.ShapeDtypeStruct((16384 * 50, DIM), jnp.float32),
    scratch_types=[
        pltpu.VMEM((CHUNK,), jnp.int32),
        pltpu.VMEM((CHUNK, DIM), jnp.float32),
        pltpu.SemaphoreType.DMA,
    ],
    compiler_params=pltpu.CompilerParams(use_tc_tiling_on_sc=False),
)
def _sc_gather(idx_hbm, table_hbm, out_hbm, idx_v, rows_v, sem):
    total = idx_hbm.shape[0]
    per_w = total // NW
    nchunk = per_w // CHUNK
    wid = lax.axis_index("s") * NUM_CORES + lax.axis_index("c")
    base = wid * per_w

    def body(g, carry):
        start = base + g * CHUNK
        pltpu.sync_copy(idx_hbm.at[pl.ds(start, CHUNK)], idx_v)
        pltpu.async_copy(table_hbm.at[idx_v], rows_v, sem).wait()
        pltpu.sync_copy(rows_v, out_hbm.at[pl.ds(start, CHUNK)])
        return carry

    lax.fori_loop(0, nchunk, body, 0)


def kernel(x, E):
    idx = x.reshape(-1).astype(jnp.int32)
    out = _sc_gather(idx, E)
    return out.reshape(x.shape + (DIM,))


# speedup vs baseline: 1.1123x; 1.0009x over previous
import functools

import jax
import jax.numpy as jnp
from jax import lax
from jax.experimental import pallas as pl
from jax.experimental.pallas import tpu as pltpu
from jax.experimental.pallas import tpu_sc as plsc

DIM = 32
NUM_CORES = 2
NUM_SUBCORES = 16
NW = NUM_CORES * NUM_SUBCORES
TOTAL = 16384 * 50
PER_W = TOTAL // NW
CHUNK = 1600
NCHUNK = PER_W // CHUNK


@functools.partial(
    pl.kernel,
    mesh=plsc.VectorSubcoreMesh(core_axis_name="c", subcore_axis_name="s"),
    out_type=jax.ShapeDtypeStruct((TOTAL, DIM), jnp.float32),
    scratch_types=[
        pltpu.VMEM((CHUNK,), jnp.int32),
        pltpu.VMEM((CHUNK,), jnp.int32),
        pltpu.VMEM((CHUNK, DIM), jnp.float32),
        pltpu.VMEM((CHUNK, DIM), jnp.float32),
        pltpu.SemaphoreType.DMA,
        pltpu.SemaphoreType.DMA,
        pltpu.SemaphoreType.DMA,
        pltpu.SemaphoreType.DMA,
        pltpu.SemaphoreType.DMA,
        pltpu.SemaphoreType.DMA,
    ],
    compiler_params=pltpu.CompilerParams(use_tc_tiling_on_sc=False),
)
def _sc_gather(idx_hbm, table_hbm, out_hbm,
               idx0, idx1, rows0, rows1,
               isem0, isem1, gsem0, gsem1, osem0, osem1):
    idx_v = (idx0, idx1)
    rows_v = (rows0, rows1)
    isem = (isem0, isem1)
    gsem = (gsem0, gsem1)
    osem = (osem0, osem1)

    wid = lax.axis_index("s") * NUM_CORES + lax.axis_index("c")
    base = wid * PER_W

    def idx_load(g):
        s = g % 2
        return pltpu.async_copy(idx_hbm.at[pl.ds(base + g * CHUNK, CHUNK)],
                                idx_v[s], isem[s])

    def gather_start(g):
        s = g % 2
        return pltpu.async_copy(table_hbm.at[idx_v[s]], rows_v[s], gsem[s])

    def store_start(g):
        s = g % 2
        return pltpu.async_copy(rows_v[s],
                                out_hbm.at[pl.ds(base + g * CHUNK, CHUNK)],
                                osem[s])

    iloads = {0: idx_load(0), 1: idx_load(1)}
    gathers = {}
    stores = {}
    iloads[0].wait()
    gathers[0] = gather_start(0)
    for g in range(NCHUNK):
        if g + 1 < NCHUNK:
            iloads[g + 1].wait()
            if g - 1 >= 0:
                stores[g - 1].wait()
            gathers[g + 1] = gather_start(g + 1)
        gathers[g].wait()
        if g + 2 < NCHUNK:
            iloads[g + 2] = idx_load(g + 2)
        stores[g] = store_start(g)
    stores[NCHUNK - 2].wait()
    stores[NCHUNK - 1].wait()


def kernel(x, E):
    idx = x.reshape(-1).astype(jnp.int32)
    out = _sc_gather(idx, E)
    return out.reshape(x.shape + (DIM,))


# speedup vs baseline: 1.8019x; 1.6200x over previous
import functools

import jax
import jax.numpy as jnp
from jax import lax
from jax.experimental import pallas as pl
from jax.experimental.pallas import tpu as pltpu
from jax.experimental.pallas import tpu_sc as plsc

DIM = 32
SEQ = 50
NUM_ROWS = 16384
NUM_CORES = 2
NUM_SUBCORES = 16
NW = NUM_CORES * NUM_SUBCORES
ROWS_PER_W = NUM_ROWS // NW
RCHUNK = 32
NCHUNK = ROWS_PER_W // RCHUNK


@functools.partial(
    pl.kernel,
    mesh=plsc.VectorSubcoreMesh(core_axis_name="c", subcore_axis_name="s"),
    out_type=jax.ShapeDtypeStruct((NUM_ROWS, SEQ, DIM), jnp.float32),
    scratch_types=[
        pltpu.VMEM((RCHUNK, SEQ), jnp.int32),
        pltpu.VMEM((RCHUNK, SEQ), jnp.int32),
        pltpu.VMEM((RCHUNK, SEQ, DIM), jnp.float32),
        pltpu.VMEM((RCHUNK, SEQ, DIM), jnp.float32),
        pltpu.SemaphoreType.DMA,
        pltpu.SemaphoreType.DMA,
        pltpu.SemaphoreType.DMA,
        pltpu.SemaphoreType.DMA,
        pltpu.SemaphoreType.DMA,
        pltpu.SemaphoreType.DMA,
    ],
    compiler_params=pltpu.CompilerParams(use_tc_tiling_on_sc=False),
)
def _sc_gather(x_hbm, table_hbm, out_hbm,
               idx0, idx1, rows0, rows1,
               isem0, isem1, gsem0, gsem1, osem0, osem1):
    idx_v = (idx0, idx1)
    rows_v = (rows0, rows1)
    isem = (isem0, isem1)
    gsem = (gsem0, gsem1)
    osem = (osem0, osem1)

    wid = lax.axis_index("s") * NUM_CORES + lax.axis_index("c")
    base = wid * ROWS_PER_W

    def idx_load(g):
        s = g % 2
        return pltpu.async_copy(x_hbm.at[pl.ds(base + g * RCHUNK, RCHUNK)],
                                idx_v[s], isem[s])

    def gather_start(g):
        s = g % 2
        return [pltpu.async_copy(table_hbm.at[idx_v[s].at[r]],
                                 rows_v[s].at[r], gsem[s])
                for r in range(RCHUNK)]

    def store_start(g):
        s = g % 2
        return pltpu.async_copy(rows_v[s],
                                out_hbm.at[pl.ds(base + g * RCHUNK, RCHUNK)],
                                osem[s])

    iloads = {0: idx_load(0), 1: idx_load(1)}
    gathers = {}
    stores = {}
    iloads[0].wait()
    gathers[0] = gather_start(0)
    for g in range(NCHUNK):
        if g + 1 < NCHUNK:
            iloads[g + 1].wait()
            if g - 1 >= 0:
                stores[g - 1].wait()
            gathers[g + 1] = gather_start(g + 1)
        for h in gathers[g]:
            h.wait()
        if g + 2 < NCHUNK:
            iloads[g + 2] = idx_load(g + 2)
        stores[g] = store_start(g)
    stores[NCHUNK - 2].wait()
    stores[NCHUNK - 1].wait()


def kernel(x, E):
    return _sc_gather(x, E)
